# Initial kernel scaffold; baseline (speedup 1.0000x reference)
import jax
import jax.numpy as jnp
from jax.experimental import pallas as pl


def kernel(feat_word, feat_topic, feat_doc, ei_ww, ei_wt, ei_wd, ei_td, ei_tt, w_ww, w_wt, w_wd, w_td, w_tt, W_ww, b_ww, W_wt, b_wt, W_wd, b_wd, W_td, b_td, W_tt, b_tt):
    raise NotImplementedError("write your pallas kernel here")


# speedup vs baseline: 1.1506x; 1.1506x over previous
import functools

import jax
import jax.numpy as jnp
from jax import lax
from jax.experimental import pallas as pl
from jax.experimental.pallas import tpu as pltpu
from jax.experimental.pallas import tpu_sc as plsc

N_WORD, N_TOPIC, N_DOC = 10000, 2000, 5000
N_WORD_P, N_TOPIC_P, N_DOC_P = 10240, 2048, 5120
IN_SIZE = 256
HALF = 128
_NT = 16
_CHUNK = 32
_EMULT = 2 * _NT * _CHUNK


def _prep_edges(ei, w, ndump):
    e = w.shape[0]
    ep = -(-e // _EMULT) * _EMULT
    pad = ep - e
    src = jnp.pad(ei[0], (0, pad))
    dst = jnp.pad(ei[1], (0, pad), constant_values=ndump)
    ew = jnp.pad(w, (0, pad))
    return src, dst, ew


def _sc_segsum(tables, etypes):
    n_et = len(etypes)
    offs = []
    tot = 0
    for e in etypes:
        offs.append(tot)
        tot += e["ndst"]
    nz = tot // _NT

    mesh = plsc.VectorSubcoreMesh(core_axis_name="c", subcore_axis_name="s")
    out_type = ([jax.ShapeDtypeStruct((e["ndst"], 2 * HALF), jnp.float32)
                 for e in etypes]
                + [jax.ShapeDtypeStruct((e["ndst"], 2 * HALF), jnp.float32)
                   for e in etypes])
    scratch_types = [
        pltpu.VMEM_SHARED((tot, HALF), jnp.float32),
        pltpu.VMEM((_CHUNK,), jnp.int32),
        pltpu.VMEM((_CHUNK,), jnp.int32),
        pltpu.VMEM((_CHUNK,), jnp.int32),
        pltpu.VMEM((_CHUNK,), jnp.float32),
        pltpu.VMEM((_CHUNK, 16), jnp.float32),
        pltpu.VMEM((_CHUNK, HALF), jnp.float32),
        pltpu.SemaphoreType.DMA,
    ]

    @functools.partial(pl.kernel, mesh=mesh, out_type=out_type,
                       scratch_types=scratch_types)
    def run(*refs):
        it = iter(refs)
        table_refs = [next(it) for _ in tables]
        ed_refs = [(next(it), next(it), next(it)) for _ in range(n_et)]
        s_refs = [next(it) for _ in range(n_et)]
        c2_refs = [next(it) for _ in range(n_et)]
        (acc, sidx, gidx, didx, ewv, w16, rows,
         sem) = [next(it) for _ in range(8)]

        c = lax.axis_index("c")
        s = lax.axis_index("s")
        wid = c * _NT + s

        def zero_acc():
            def zbuf_body(i, _):
                zero16 = jnp.zeros((16,), jnp.float32)
                for j in range(HALF // 16):
                    rows[i, pl.ds(j * 16, 16)] = zero16
                return 0

            lax.fori_loop(0, _CHUNK, zbuf_body, 0)

            def zacc_body(t, _):
                r0 = s * nz + t * _CHUNK
                pltpu.sync_copy(rows, acc.at[pl.ds(r0, _CHUNK)])
                return 0

            lax.fori_loop(0, nz // _CHUNK, zacc_body, 0)

        zero_acc()
        plsc.subcore_barrier()

        for idx_et in range(n_et):
            src_r, dst_r, ew_r = ed_refs[idx_et]
            tab_r = table_refs[etypes[idx_et]["ti"]]
            off = offs[idx_et]
            ept = etypes[idx_et]["src"].shape[0] // _NT
            nchunks = ept // _CHUNK

            def chunk_body(g, _, src_r=src_r, dst_r=dst_r, ew_r=ew_r,
                           tab_r=tab_r, off=off, ept=ept):
                eb = s * ept + g * _CHUNK
                pltpu.sync_copy(src_r.at[pl.ds(eb, _CHUNK)], sidx)
                pltpu.sync_copy(dst_r.at[pl.ds(eb, _CHUNK)], didx)
                pltpu.sync_copy(ew_r.at[pl.ds(eb, _CHUNK)], ewv)

                def idx_body(j, _):
                    sl = pl.ds(j * 16, 16)
                    gidx[sl] = sidx[sl] * 2 + c
                    didx[sl] = didx[sl] + off
                    return 0

                lax.fori_loop(0, _CHUNK // 16, idx_body, 0)
                pltpu.async_copy(tab_r.at[gidx], rows, sem).wait()

                def wexp_body(g2, _):
                    wv = ewv[pl.ds(g2 * 16, 16)]
                    for k in range(16):
                        w16[g2 * 16 + k, :] = jnp.broadcast_to(wv[k], (16,))
                    return 0

                lax.fori_loop(0, _CHUNK // 16, wexp_body, 0)

                def mul_body(i, _):
                    wv = w16[i, :]
                    for j in range(HALF // 16):
                        sl = pl.ds(j * 16, 16)
                        rows[i, sl] = rows[i, sl] * wv
                    return 0

                lax.fori_loop(0, _CHUNK, mul_body, 0)
                pltpu.sync_copy(rows, acc.at[didx], add=True)
                return 0

            lax.fori_loop(0, nchunks, chunk_body, 0)

        plsc.subcore_barrier()

        for idx_et in range(n_et):
            nrf = etypes[idx_et]["ndst"] // _NT
            off = offs[idx_et]
            s_r = s_refs[idx_et]

            def flush_body(t, _, nrf=nrf, off=off, s_r=s_r):
                r0 = s * nrf + t * _CHUNK
                pltpu.sync_copy(acc.at[pl.ds(off + r0, _CHUNK)], rows)
                pltpu.sync_copy(
                    rows, s_r.at[pl.ds(r0, _CHUNK), pl.ds(c * HALF, HALF)])
                return 0

            lax.fori_loop(0, nrf // _CHUNK, flush_body, 0)

        plsc.subcore_barrier()
        zero_acc()
        plsc.subcore_barrier()

        for idx_et in range(n_et):
            _, dst_r, ew_r = ed_refs[idx_et]
            off = offs[idx_et]
            ept2 = etypes[idx_et]["src"].shape[0] // (2 * _NT)
            nchunks2 = ept2 // _CHUNK

            def cnt_body(g, _, dst_r=dst_r, ew_r=ew_r, off=off, ept2=ept2):
                eb = wid * ept2 + g * _CHUNK
                pltpu.sync_copy(dst_r.at[pl.ds(eb, _CHUNK)], didx)
                pltpu.sync_copy(ew_r.at[pl.ds(eb, _CHUNK)], ewv)

                def idx_body(j, _):
                    sl = pl.ds(j * 16, 16)
                    didx[sl] = didx[sl] + off
                    return 0

                lax.fori_loop(0, _CHUNK // 16, idx_body, 0)

                def cexp_body(g2, _):
                    lane = lax.iota(jnp.int32, 16)
                    sel0 = (1 - jnp.minimum(jnp.abs(lane), 1)).astype(jnp.float32)
                    sel1 = (1 - jnp.minimum(jnp.abs(lane - 1), 1)).astype(jnp.float32)
                    wv = ewv[pl.ds(g2 * 16, 16)]
                    for k in range(16):
                        wsp = jnp.broadcast_to(wv[k], (16,))
                        rows[g2 * 16 + k, pl.ds(0, 16)] = wsp * sel0 + sel1
                    return 0

                lax.fori_loop(0, _CHUNK // 16, cexp_body, 0)
                pltpu.sync_copy(rows, acc.at[didx], add=True)
                return 0

            lax.fori_loop(0, nchunks2, cnt_body, 0)

        plsc.subcore_barrier()

        for idx_et in range(n_et):
            nrf = etypes[idx_et]["ndst"] // _NT
            off = offs[idx_et]
            c2_r = c2_refs[idx_et]

            def cflush_body(t, _, nrf=nrf, off=off, c2_r=c2_r):
                r0 = s * nrf + t * _CHUNK
                pltpu.sync_copy(acc.at[pl.ds(off + r0, _CHUNK)], rows)
                pltpu.sync_copy(
                    rows, c2_r.at[pl.ds(r0, _CHUNK), pl.ds(c * HALF, HALF)])
                return 0

            lax.fori_loop(0, nrf // _CHUNK, cflush_body, 0)

    args = list(tables)
    for e in etypes:
        args += [e["src"], e["dst"], e["ew"]]
    outs = run(*args)
    return outs[:n_et], outs[n_et:]


def _scale_part(S, c2, W, b):
    wsum = c2[:, 0] + c2[:, HALF]
    cnt = jnp.maximum(c2[:, 1] + c2[:, HALF + 1], 1.0)
    inv = 1.0 / cnt
    h = jnp.dot(S * inv[:, None], W, preferred_element_type=jnp.float32)
    return h + (wsum * inv)[:, None] * b[None, :]


def _tc_combine1(S, c2, W, b):
    def body(s_ref, c2_ref, w_ref, b_ref, o_ref):
        o_ref[...] = _scale_part(s_ref[...], c2_ref[...], w_ref[...], b_ref[...])

    return pl.pallas_call(
        body, out_shape=jax.ShapeDtypeStruct((S.shape[0], IN_SIZE), jnp.float32),
    )(S, c2, W, b)


def _tc_combine2(Sa, c2a, Wa, ba, Sb, c2b, Wb, bb):
    def body(sa, ca, wa, ba_, sb, cb, wb, bb_, o_ref):
        o_ref[...] = (_scale_part(sa[...], ca[...], wa[...], ba_[...])
                      + _scale_part(sb[...], cb[...], wb[...], bb_[...]))

    return pl.pallas_call(
        body, out_shape=jax.ShapeDtypeStruct((Sa.shape[0], IN_SIZE), jnp.float32),
    )(Sa, c2a, Wa, ba, Sb, c2b, Wb, bb)


def kernel(feat_word, feat_topic, feat_doc,
           ei_ww, ei_wt, ei_wd, ei_td, ei_tt,
           w_ww, w_wt, w_wd, w_td, w_tt,
           W_ww, b_ww, W_wt, b_wt, W_wd, b_wd, W_td, b_td, W_tt, b_tt):
    word2 = feat_word.reshape(-1, HALF)
    topic2 = feat_topic.reshape(-1, HALF)

    s_ww, d_ww, e_ww = _prep_edges(ei_ww, w_ww, N_WORD)
    s_wt, d_wt, e_wt = _prep_edges(ei_wt, w_wt, N_TOPIC)
    s_wd, d_wd, e_wd = _prep_edges(ei_wd, w_wd, N_DOC)
    s_td, d_td, e_td = _prep_edges(ei_td, w_td, N_DOC)
    s_tt, d_tt, e_tt = _prep_edges(ei_tt, w_tt, N_TOPIC)

    (S_a, C_a) = _sc_segsum(
        [word2, topic2],
        [dict(ti=0, src=s_ww, dst=d_ww, ew=e_ww, ndst=N_WORD_P),
         dict(ti=1, src=s_tt, dst=d_tt, ew=e_tt, ndst=N_TOPIC_P)])
    S_ww_sum, S_tt_sum = S_a
    c2_ww, c2_tt = C_a

    h_word_p = _tc_combine1(S_ww_sum, c2_ww, W_ww, b_ww)
    hword2 = h_word_p.reshape(-1, HALF)
    h_word = h_word_p[:N_WORD]

    (S_c, C_c) = _sc_segsum(
        [hword2, topic2],
        [dict(ti=0, src=s_wt, dst=d_wt, ew=e_wt, ndst=N_TOPIC_P),
         dict(ti=0, src=s_wd, dst=d_wd, ew=e_wd, ndst=N_DOC_P),
         dict(ti=1, src=s_td, dst=d_td, ew=e_td, ndst=N_DOC_P)])
    S_wt_sum, S_wd_sum, S_td_sum = S_c
    c2_wt, c2_wd, c2_td = C_c

    h_topic = _tc_combine2(S_wt_sum, c2_wt, W_wt, b_wt,
                           S_tt_sum, c2_tt, W_tt, b_tt)[:N_TOPIC]
    h_doc = _tc_combine2(S_wd_sum, c2_wd, W_wd, b_wd,
                         S_td_sum, c2_td, W_td, b_td)[:N_DOC]
    return (h_word, h_topic, h_doc)


# speedup vs baseline: 1.5255x; 1.3258x over previous
import functools

import jax
import jax.numpy as jnp
from jax import lax
from jax.experimental import pallas as pl
from jax.experimental.pallas import tpu as pltpu
from jax.experimental.pallas import tpu_sc as plsc

N_WORD, N_TOPIC, N_DOC = 10000, 2000, 5000
N_WORD_P, N_TOPIC_P, N_DOC_P = 10240, 2048, 5120
IN_SIZE = 256
HALF = 128
_NT = 16
_CHUNK = 32
_EMULT = 2 * _NT * _CHUNK


def _prep_edges(ei, w, ndump):
    e = w.shape[0]
    ep = -(-e // _EMULT) * _EMULT
    pad = ep - e
    src = jnp.pad(ei[0], (0, pad))
    dst = jnp.pad(ei[1], (0, pad), constant_values=ndump)
    ew = jnp.pad(w, (0, pad))
    return src, dst, ew


def _sc_segsum(tables, etypes):
    n_et = len(etypes)
    offs = []
    tot = 0
    for e in etypes:
        offs.append(tot)
        tot += e["ndst"]
    nz = tot // _NT

    mesh = plsc.VectorSubcoreMesh(core_axis_name="c", subcore_axis_name="s")
    out_type = ([jax.ShapeDtypeStruct((e["ndst"], 2 * HALF), jnp.float32)
                 for e in etypes]
                + [jax.ShapeDtypeStruct((e["ndst"], 2 * HALF), jnp.float32)
                   for e in etypes])
    scratch_types = [
        pltpu.VMEM_SHARED((tot, HALF), jnp.float32),
        pltpu.VMEM((_CHUNK,), jnp.int32),
        pltpu.VMEM((_CHUNK,), jnp.int32),
        pltpu.VMEM((_CHUNK,), jnp.int32),
        pltpu.VMEM((_CHUNK,), jnp.float32),
        pltpu.VMEM((_CHUNK,), jnp.int32),
        pltpu.VMEM((_CHUNK,), jnp.int32),
        pltpu.VMEM((_CHUNK,), jnp.float32),
        pltpu.VMEM((_CHUNK, 16), jnp.float32),
        pltpu.VMEM((_CHUNK, HALF), jnp.float32),
        pltpu.VMEM((_CHUNK, HALF), jnp.float32),
        pltpu.SemaphoreType.DMA,
        pltpu.SemaphoreType.DMA,
    ]

    @functools.partial(pl.kernel, mesh=mesh, out_type=out_type,
                       scratch_types=scratch_types)
    def run(*refs):
        it = iter(refs)
        table_refs = [next(it) for _ in tables]
        ed_refs = [(next(it), next(it), next(it)) for _ in range(n_et)]
        s_refs = [next(it) for _ in range(n_et)]
        c2_refs = [next(it) for _ in range(n_et)]
        (acc, sidx, gidx0, didx0, ewv0, gidx1, didx1, ewv1, w16,
         rows0, rows1, sem0, sem1) = [next(it) for _ in range(13)]
        gidx, didx, ewv, rows = gidx0, didx0, ewv0, rows0
        slots = ((gidx0, didx0, ewv0, rows0, sem0),
                 (gidx1, didx1, ewv1, rows1, sem1))

        c = lax.axis_index("c")
        s = lax.axis_index("s")
        wid = c * _NT + s

        def zero_acc():
            def zbuf_body(i, _):
                zero16 = jnp.zeros((16,), jnp.float32)
                for j in range(HALF // 16):
                    rows[i, pl.ds(j * 16, 16)] = zero16
                return 0

            lax.fori_loop(0, _CHUNK, zbuf_body, 0)

            def zacc_body(t, _):
                r0 = s * nz + t * _CHUNK
                pltpu.sync_copy(rows, acc.at[pl.ds(r0, _CHUNK)])
                return 0

            lax.fori_loop(0, nz // _CHUNK, zacc_body, 0)

        zero_acc()
        plsc.subcore_barrier()

        def load_idx(slot, gch, src_r, dst_r, ew_r, off, ept):
            gi, di, ev, _, _ = slot
            eb = s * ept + gch * _CHUNK
            pltpu.sync_copy(src_r.at[pl.ds(eb, _CHUNK)], sidx)
            pltpu.sync_copy(dst_r.at[pl.ds(eb, _CHUNK)], di)
            pltpu.sync_copy(ew_r.at[pl.ds(eb, _CHUNK)], ev)

            def idx_body(j, _):
                sl = pl.ds(j * 16, 16)
                gi[sl] = sidx[sl] * 2 + c
                di[sl] = di[sl] + off
                return 0

            lax.fori_loop(0, _CHUNK // 16, idx_body, 0)

        def start_gather(slot, tab_r):
            gi, _, _, rr, sm = slot
            pltpu.make_async_copy(tab_r.at[gi], rr, sm).start()

        def process(slot, tab_r):
            gi, di, ev, rr, sm = slot
            pltpu.make_async_copy(tab_r.at[gi], rr, sm).wait()

            def wexp_body(g2, _):
                wv = ev[pl.ds(g2 * 16, 16)]
                for k in range(16):
                    w16[g2 * 16 + k, :] = jnp.broadcast_to(wv[k], (16,))
                return 0

            lax.fori_loop(0, _CHUNK // 16, wexp_body, 0)

            def mul_body(i, _):
                wv = w16[i, :]
                for j in range(HALF // 16):
                    sl = pl.ds(j * 16, 16)
                    rr[i, sl] = rr[i, sl] * wv
                return 0

            lax.fori_loop(0, _CHUNK, mul_body, 0)
            pltpu.sync_copy(rr, acc.at[di], add=True)

        for idx_et in range(n_et):
            src_r, dst_r, ew_r = ed_refs[idx_et]
            tab_r = table_refs[etypes[idx_et]["ti"]]
            off = offs[idx_et]
            ept = etypes[idx_et]["src"].shape[0] // _NT
            nchunks = ept // _CHUNK

            load_idx(slots[0], 0, src_r, dst_r, ew_r, off, ept)
            start_gather(slots[0], tab_r)

            def pair_body(gg, _, src_r=src_r, dst_r=dst_r, ew_r=ew_r,
                          tab_r=tab_r, off=off, ept=ept, nchunks=nchunks):
                load_idx(slots[1], 2 * gg + 1, src_r, dst_r, ew_r, off, ept)
                start_gather(slots[1], tab_r)
                process(slots[0], tab_r)
                g2 = jnp.minimum(2 * gg + 2, nchunks - 1)
                load_idx(slots[0], g2, src_r, dst_r, ew_r, off, ept)
                start_gather(slots[0], tab_r)
                process(slots[1], tab_r)
                return 0

            lax.fori_loop(0, nchunks // 2, pair_body, 0)
            pltpu.make_async_copy(tab_r.at[gidx0], rows0, sem0).wait()

        plsc.subcore_barrier()

        for idx_et in range(n_et):
            nrf = etypes[idx_et]["ndst"] // _NT
            off = offs[idx_et]
            s_r = s_refs[idx_et]

            def flush_body(t, _, nrf=nrf, off=off, s_r=s_r):
                r0 = s * nrf + t * _CHUNK
                pltpu.sync_copy(acc.at[pl.ds(off + r0, _CHUNK)], rows)
                pltpu.sync_copy(
                    rows, s_r.at[pl.ds(r0, _CHUNK), pl.ds(c * HALF, HALF)])
                return 0

            lax.fori_loop(0, nrf // _CHUNK, flush_body, 0)

        plsc.subcore_barrier()
        zero_acc()
        plsc.subcore_barrier()

        for idx_et in range(n_et):
            _, dst_r, ew_r = ed_refs[idx_et]
            off = offs[idx_et]
            ept2 = etypes[idx_et]["src"].shape[0] // (2 * _NT)
            nchunks2 = ept2 // _CHUNK

            def cnt_body(g, _, dst_r=dst_r, ew_r=ew_r, off=off, ept2=ept2):
                eb = wid * ept2 + g * _CHUNK
                pltpu.sync_copy(dst_r.at[pl.ds(eb, _CHUNK)], didx)
                pltpu.sync_copy(ew_r.at[pl.ds(eb, _CHUNK)], ewv)

                def idx_body(j, _):
                    sl = pl.ds(j * 16, 16)
                    didx[sl] = didx[sl] + off
                    return 0

                lax.fori_loop(0, _CHUNK // 16, idx_body, 0)

                def cexp_body(g2, _):
                    lane = lax.iota(jnp.int32, 16)
                    sel0 = (1 - jnp.minimum(jnp.abs(lane), 1)).astype(jnp.float32)
                    sel1 = (1 - jnp.minimum(jnp.abs(lane - 1), 1)).astype(jnp.float32)
                    wv = ewv[pl.ds(g2 * 16, 16)]
                    for k in range(16):
                        wsp = jnp.broadcast_to(wv[k], (16,))
                        rows[g2 * 16 + k, pl.ds(0, 16)] = wsp * sel0 + sel1
                    return 0

                lax.fori_loop(0, _CHUNK // 16, cexp_body, 0)
                pltpu.sync_copy(rows, acc.at[didx], add=True)
                return 0

            lax.fori_loop(0, nchunks2, cnt_body, 0)

        plsc.subcore_barrier()

        for idx_et in range(n_et):
            nrf = etypes[idx_et]["ndst"] // _NT
            off = offs[idx_et]
            c2_r = c2_refs[idx_et]

            def cflush_body(t, _, nrf=nrf, off=off, c2_r=c2_r):
                r0 = s * nrf + t * _CHUNK
                pltpu.sync_copy(acc.at[pl.ds(off + r0, _CHUNK)], rows)
                pltpu.sync_copy(
                    rows, c2_r.at[pl.ds(r0, _CHUNK), pl.ds(c * HALF, HALF)])
                return 0

            lax.fori_loop(0, nrf // _CHUNK, cflush_body, 0)

    args = list(tables)
    for e in etypes:
        args += [e["src"], e["dst"], e["ew"]]
    outs = run(*args)
    return outs[:n_et], outs[n_et:]


def _scale_part(S, c2, W, b):
    wsum = c2[:, 0] + c2[:, HALF]
    cnt = jnp.maximum(c2[:, 1] + c2[:, HALF + 1], 1.0)
    inv = 1.0 / cnt
    h = jnp.dot(S * inv[:, None], W, preferred_element_type=jnp.float32)
    return h + (wsum * inv)[:, None] * b[None, :]


def _tc_combine1(S, c2, W, b):
    def body(s_ref, c2_ref, w_ref, b_ref, o_ref):
        o_ref[...] = _scale_part(s_ref[...], c2_ref[...], w_ref[...], b_ref[...])

    return pl.pallas_call(
        body, out_shape=jax.ShapeDtypeStruct((S.shape[0], IN_SIZE), jnp.float32),
    )(S, c2, W, b)


def _tc_combine2(Sa, c2a, Wa, ba, Sb, c2b, Wb, bb):
    def body(sa, ca, wa, ba_, sb, cb, wb, bb_, o_ref):
        o_ref[...] = (_scale_part(sa[...], ca[...], wa[...], ba_[...])
                      + _scale_part(sb[...], cb[...], wb[...], bb_[...]))

    return pl.pallas_call(
        body, out_shape=jax.ShapeDtypeStruct((Sa.shape[0], IN_SIZE), jnp.float32),
    )(Sa, c2a, Wa, ba, Sb, c2b, Wb, bb)


def kernel(feat_word, feat_topic, feat_doc,
           ei_ww, ei_wt, ei_wd, ei_td, ei_tt,
           w_ww, w_wt, w_wd, w_td, w_tt,
           W_ww, b_ww, W_wt, b_wt, W_wd, b_wd, W_td, b_td, W_tt, b_tt):
    word2 = feat_word.reshape(-1, HALF)
    topic2 = feat_topic.reshape(-1, HALF)

    s_ww, d_ww, e_ww = _prep_edges(ei_ww, w_ww, N_WORD)
    s_wt, d_wt, e_wt = _prep_edges(ei_wt, w_wt, N_TOPIC)
    s_wd, d_wd, e_wd = _prep_edges(ei_wd, w_wd, N_DOC)
    s_td, d_td, e_td = _prep_edges(ei_td, w_td, N_DOC)
    s_tt, d_tt, e_tt = _prep_edges(ei_tt, w_tt, N_TOPIC)

    (S_a, C_a) = _sc_segsum(
        [word2, topic2],
        [dict(ti=0, src=s_ww, dst=d_ww, ew=e_ww, ndst=N_WORD_P),
         dict(ti=1, src=s_tt, dst=d_tt, ew=e_tt, ndst=N_TOPIC_P)])
    S_ww_sum, S_tt_sum = S_a
    c2_ww, c2_tt = C_a

    h_word_p = _tc_combine1(S_ww_sum, c2_ww, W_ww, b_ww)
    hword2 = h_word_p.reshape(-1, HALF)
    h_word = h_word_p[:N_WORD]

    (S_c, C_c) = _sc_segsum(
        [hword2, topic2],
        [dict(ti=0, src=s_wt, dst=d_wt, ew=e_wt, ndst=N_TOPIC_P),
         dict(ti=0, src=s_wd, dst=d_wd, ew=e_wd, ndst=N_DOC_P),
         dict(ti=1, src=s_td, dst=d_td, ew=e_td, ndst=N_DOC_P)])
    S_wt_sum, S_wd_sum, S_td_sum = S_c
    c2_wt, c2_wd, c2_td = C_c

    h_topic = _tc_combine2(S_wt_sum, c2_wt, W_wt, b_wt,
                           S_tt_sum, c2_tt, W_tt, b_tt)[:N_TOPIC]
    h_doc = _tc_combine2(S_wd_sum, c2_wd, W_wd, b_wd,
                         S_td_sum, c2_td, W_td, b_td)[:N_DOC]
    return (h_word, h_topic, h_doc)


# speedup vs baseline: 2.2050x; 1.4455x over previous
import functools

import jax
import jax.numpy as jnp
from jax import lax
from jax.experimental import pallas as pl
from jax.experimental.pallas import tpu as pltpu
from jax.experimental.pallas import tpu_sc as plsc

N_WORD, N_TOPIC, N_DOC = 10000, 2000, 5000
N_WORD_P, N_TOPIC_P, N_DOC_P = 10240, 2048, 5120
IN_SIZE = 256
HALF = 128
_NT = 16
_CHUNK = 64
_EMULT = 2 * _NT * _CHUNK


def _prep_edges(ei, w, ndump):
    e = w.shape[0]
    ep = -(-e // _EMULT) * _EMULT
    pad = ep - e
    src = jnp.pad(ei[0], (0, pad))
    dst = jnp.pad(ei[1], (0, pad), constant_values=ndump)
    ew = jnp.pad(w, (0, pad))
    return src, dst, ew


def _sc_segsum(tables, etypes):
    n_et = len(etypes)
    offs = []
    tot = 0
    for e in etypes:
        offs.append(tot)
        tot += e["ndst"]
    nz = tot // _NT

    mesh = plsc.VectorSubcoreMesh(core_axis_name="c", subcore_axis_name="s")
    out_type = ([jax.ShapeDtypeStruct((e["ndst"], 2 * HALF), jnp.float32)
                 for e in etypes]
                + [jax.ShapeDtypeStruct((e["ndst"], 2 * HALF), jnp.float32)
                   for e in etypes])
    scratch_types = [
        pltpu.VMEM_SHARED((tot, HALF), jnp.float32),
        pltpu.VMEM((_CHUNK,), jnp.int32),
        pltpu.VMEM((_CHUNK,), jnp.int32),
        pltpu.VMEM((_CHUNK,), jnp.int32),
        pltpu.VMEM((_CHUNK,), jnp.float32),
        pltpu.VMEM((_CHUNK,), jnp.int32),
        pltpu.VMEM((_CHUNK,), jnp.int32),
        pltpu.VMEM((_CHUNK,), jnp.float32),
        pltpu.VMEM((_CHUNK, 16), jnp.float32),
        pltpu.VMEM((_CHUNK, HALF), jnp.float32),
        pltpu.VMEM((_CHUNK, HALF), jnp.float32),
        pltpu.SemaphoreType.DMA,
        pltpu.SemaphoreType.DMA,
    ]

    @functools.partial(pl.kernel, mesh=mesh, out_type=out_type,
                       scratch_types=scratch_types)
    def run(*refs):
        it = iter(refs)
        table_refs = [next(it) for _ in tables]
        ed_refs = [(next(it), next(it), next(it)) for _ in range(n_et)]
        s_refs = [next(it) for _ in range(n_et)]
        c2_refs = [next(it) for _ in range(n_et)]
        (acc, sidx, gidx0, didx0, ewv0, gidx1, didx1, ewv1, w16,
         rows0, rows1, sem0, sem1) = [next(it) for _ in range(13)]
        gidx, didx, ewv, rows = gidx0, didx0, ewv0, rows0
        slots = ((gidx0, didx0, ewv0, rows0, sem0),
                 (gidx1, didx1, ewv1, rows1, sem1))

        c = lax.axis_index("c")
        s = lax.axis_index("s")
        wid = c * _NT + s

        def zero_acc():
            def zbuf_body(i, _):
                zero16 = jnp.zeros((16,), jnp.float32)
                for j in range(HALF // 16):
                    rows[i, pl.ds(j * 16, 16)] = zero16
                return 0

            lax.fori_loop(0, _CHUNK, zbuf_body, 0)

            def zacc_body(t, _):
                r0 = s * nz + t * _CHUNK
                pltpu.sync_copy(rows, acc.at[pl.ds(r0, _CHUNK)])
                return 0

            lax.fori_loop(0, nz // _CHUNK, zacc_body, 0)

        zero_acc()
        plsc.subcore_barrier()

        def load_idx(slot, gch, src_r, dst_r, ew_r, off, ept):
            gi, di, ev, _, _ = slot
            eb = s * ept + gch * _CHUNK
            pltpu.sync_copy(src_r.at[pl.ds(eb, _CHUNK)], sidx)
            pltpu.sync_copy(dst_r.at[pl.ds(eb, _CHUNK)], di)
            pltpu.sync_copy(ew_r.at[pl.ds(eb, _CHUNK)], ev)

            def idx_body(j, _):
                sl = pl.ds(j * 16, 16)
                gi[sl] = sidx[sl] * 2 + c
                di[sl] = di[sl] + off
                return 0

            lax.fori_loop(0, _CHUNK // 16, idx_body, 0)

        def start_gather(slot, tab_r):
            gi, _, _, rr, sm = slot
            pltpu.make_async_copy(tab_r.at[gi], rr, sm).start()

        def process(slot, tab_r):
            gi, di, ev, rr, sm = slot
            pltpu.make_async_copy(tab_r.at[gi], rr, sm).wait()

            def wexp_body(g2, _):
                wv = ev[pl.ds(g2 * 16, 16)]
                for k in range(16):
                    w16[g2 * 16 + k, :] = jnp.broadcast_to(wv[k], (16,))
                return 0

            lax.fori_loop(0, _CHUNK // 16, wexp_body, 0)

            def mul_body(i, _):
                wv = w16[i, :]
                for j in range(HALF // 16):
                    sl = pl.ds(j * 16, 16)
                    rr[i, sl] = rr[i, sl] * wv
                return 0

            lax.fori_loop(0, _CHUNK, mul_body, 0)
            pltpu.sync_copy(rr, acc.at[di], add=True)

        for idx_et in range(n_et):
            src_r, dst_r, ew_r = ed_refs[idx_et]
            tab_r = table_refs[etypes[idx_et]["ti"]]
            off = offs[idx_et]
            ept = etypes[idx_et]["src"].shape[0] // _NT
            nchunks = ept // _CHUNK

            load_idx(slots[0], 0, src_r, dst_r, ew_r, off, ept)
            start_gather(slots[0], tab_r)

            def pair_body(gg, _, src_r=src_r, dst_r=dst_r, ew_r=ew_r,
                          tab_r=tab_r, off=off, ept=ept, nchunks=nchunks):
                load_idx(slots[1], 2 * gg + 1, src_r, dst_r, ew_r, off, ept)
                start_gather(slots[1], tab_r)
                process(slots[0], tab_r)
                g2 = jnp.minimum(2 * gg + 2, nchunks - 1)
                load_idx(slots[0], g2, src_r, dst_r, ew_r, off, ept)
                start_gather(slots[0], tab_r)
                process(slots[1], tab_r)
                return 0

            lax.fori_loop(0, nchunks // 2, pair_body, 0)
            pltpu.make_async_copy(tab_r.at[gidx0], rows0, sem0).wait()

        plsc.subcore_barrier()

        for idx_et in range(n_et):
            nrf = etypes[idx_et]["ndst"] // _NT
            off = offs[idx_et]
            s_r = s_refs[idx_et]

            def flush_body(t, _, nrf=nrf, off=off, s_r=s_r):
                r0 = s * nrf + t * _CHUNK
                pltpu.sync_copy(acc.at[pl.ds(off + r0, _CHUNK)], rows)
                pltpu.sync_copy(
                    rows, s_r.at[pl.ds(r0, _CHUNK), pl.ds(c * HALF, HALF)])
                return 0

            lax.fori_loop(0, nrf // _CHUNK, flush_body, 0)

        plsc.subcore_barrier()
        zero_acc()
        plsc.subcore_barrier()

        for idx_et in range(n_et):
            _, dst_r, ew_r = ed_refs[idx_et]
            off = offs[idx_et]
            ept2 = etypes[idx_et]["src"].shape[0] // (2 * _NT)
            nchunks2 = ept2 // _CHUNK

            def cnt_body(g, _, dst_r=dst_r, ew_r=ew_r, off=off, ept2=ept2):
                eb = wid * ept2 + g * _CHUNK
                pltpu.sync_copy(dst_r.at[pl.ds(eb, _CHUNK)], didx)
                pltpu.sync_copy(ew_r.at[pl.ds(eb, _CHUNK)], ewv)

                def idx_body(j, _):
                    sl = pl.ds(j * 16, 16)
                    didx[sl] = didx[sl] + off
                    return 0

                lax.fori_loop(0, _CHUNK // 16, idx_body, 0)

                def cexp_body(g2, _):
                    lane = lax.iota(jnp.int32, 16)
                    sel0 = (1 - jnp.minimum(jnp.abs(lane), 1)).astype(jnp.float32)
                    sel1 = (1 - jnp.minimum(jnp.abs(lane - 1), 1)).astype(jnp.float32)
                    wv = ewv[pl.ds(g2 * 16, 16)]
                    for k in range(16):
                        wsp = jnp.broadcast_to(wv[k], (16,))
                        rows[g2 * 16 + k, pl.ds(0, 16)] = wsp * sel0 + sel1
                    return 0

                lax.fori_loop(0, _CHUNK // 16, cexp_body, 0)
                pltpu.sync_copy(rows, acc.at[didx], add=True)
                return 0

            lax.fori_loop(0, nchunks2, cnt_body, 0)

        plsc.subcore_barrier()

        for idx_et in range(n_et):
            nrf = etypes[idx_et]["ndst"] // _NT
            off = offs[idx_et]
            c2_r = c2_refs[idx_et]

            def cflush_body(t, _, nrf=nrf, off=off, c2_r=c2_r):
                r0 = s * nrf + t * _CHUNK
                pltpu.sync_copy(acc.at[pl.ds(off + r0, _CHUNK)], rows)
                pltpu.sync_copy(
                    rows, c2_r.at[pl.ds(r0, _CHUNK), pl.ds(c * HALF, HALF)])
                return 0

            lax.fori_loop(0, nrf // _CHUNK, cflush_body, 0)

    args = list(tables)
    for e in etypes:
        args += [e["src"], e["dst"], e["ew"]]
    outs = run(*args)
    return outs[:n_et], outs[n_et:]


def _scale_part(S, c2, W, b):
    wsum = c2[:, 0] + c2[:, HALF]
    cnt = jnp.maximum(c2[:, 1] + c2[:, HALF + 1], 1.0)
    inv = 1.0 / cnt
    h = jnp.dot(S * inv[:, None], W, preferred_element_type=jnp.float32)
    return h + (wsum * inv)[:, None] * b[None, :]


def _tc_combine1(S, c2, W, b):
    def body(s_ref, c2_ref, w_ref, b_ref, o_ref):
        o_ref[...] = _scale_part(s_ref[...], c2_ref[...], w_ref[...], b_ref[...])

    return pl.pallas_call(
        body, out_shape=jax.ShapeDtypeStruct((S.shape[0], IN_SIZE), jnp.float32),
    )(S, c2, W, b)


def _tc_combine2(Sa, c2a, Wa, ba, Sb, c2b, Wb, bb):
    def body(sa, ca, wa, ba_, sb, cb, wb, bb_, o_ref):
        o_ref[...] = (_scale_part(sa[...], ca[...], wa[...], ba_[...])
                      + _scale_part(sb[...], cb[...], wb[...], bb_[...]))

    return pl.pallas_call(
        body, out_shape=jax.ShapeDtypeStruct((Sa.shape[0], IN_SIZE), jnp.float32),
    )(Sa, c2a, Wa, ba, Sb, c2b, Wb, bb)


def kernel(feat_word, feat_topic, feat_doc,
           ei_ww, ei_wt, ei_wd, ei_td, ei_tt,
           w_ww, w_wt, w_wd, w_td, w_tt,
           W_ww, b_ww, W_wt, b_wt, W_wd, b_wd, W_td, b_td, W_tt, b_tt):
    word2 = feat_word.reshape(-1, HALF)
    topic2 = feat_topic.reshape(-1, HALF)

    s_ww, d_ww, e_ww = _prep_edges(ei_ww, w_ww, N_WORD)
    s_wt, d_wt, e_wt = _prep_edges(ei_wt, w_wt, N_TOPIC)
    s_wd, d_wd, e_wd = _prep_edges(ei_wd, w_wd, N_DOC)
    s_td, d_td, e_td = _prep_edges(ei_td, w_td, N_DOC)
    s_tt, d_tt, e_tt = _prep_edges(ei_tt, w_tt, N_TOPIC)

    (S_a, C_a) = _sc_segsum(
        [word2, topic2],
        [dict(ti=0, src=s_ww, dst=d_ww, ew=e_ww, ndst=N_WORD_P),
         dict(ti=1, src=s_tt, dst=d_tt, ew=e_tt, ndst=N_TOPIC_P)])
    S_ww_sum, S_tt_sum = S_a
    c2_ww, c2_tt = C_a

    h_word_p = _tc_combine1(S_ww_sum, c2_ww, W_ww, b_ww)
    hword2 = h_word_p.reshape(-1, HALF)
    h_word = h_word_p[:N_WORD]

    (S_c, C_c) = _sc_segsum(
        [hword2, topic2],
        [dict(ti=0, src=s_wt, dst=d_wt, ew=e_wt, ndst=N_TOPIC_P),
         dict(ti=0, src=s_wd, dst=d_wd, ew=e_wd, ndst=N_DOC_P),
         dict(ti=1, src=s_td, dst=d_td, ew=e_td, ndst=N_DOC_P)])
    S_wt_sum, S_wd_sum, S_td_sum = S_c
    c2_wt, c2_wd, c2_td = C_c

    h_topic = _tc_combine2(S_wt_sum, c2_wt, W_wt, b_wt,
                           S_tt_sum, c2_tt, W_tt, b_tt)[:N_TOPIC]
    h_doc = _tc_combine2(S_wd_sum, c2_wd, W_wd, b_wd,
                         S_td_sum, c2_td, W_td, b_td)[:N_DOC]
    return (h_word, h_topic, h_doc)


# speedup vs baseline: 2.2625x; 1.0260x over previous
import functools

import jax
import jax.numpy as jnp
from jax import lax
from jax.experimental import pallas as pl
from jax.experimental.pallas import tpu as pltpu
from jax.experimental.pallas import tpu_sc as plsc

N_WORD, N_TOPIC, N_DOC = 10000, 2000, 5000
N_WORD_P, N_TOPIC_P, N_DOC_P = 10240, 2048, 5120
IN_SIZE = 256
HALF = 128
_NT = 16
_CHUNK = 64
_EMULT = 2 * _NT * _CHUNK


def _prep_edges(ei, w, ndump):
    e = w.shape[0]
    ep = -(-e // _EMULT) * _EMULT
    pad = ep - e
    src = jnp.pad(ei[0], (0, pad))
    dst = jnp.pad(ei[1], (0, pad), constant_values=ndump)
    ew = jnp.pad(w, (0, pad))
    return src, dst, ew


def _sc_segsum(tables, etypes):
    n_et = len(etypes)
    offs = []
    tot = 0
    for e in etypes:
        offs.append(tot)
        tot += e["ndst"]
    nz = tot // _NT

    mesh = plsc.VectorSubcoreMesh(core_axis_name="c", subcore_axis_name="s")
    out_type = ([jax.ShapeDtypeStruct((e["ndst"], 2 * HALF), jnp.float32)
                 for e in etypes]
                + [jax.ShapeDtypeStruct((e["ndst"], 2 * HALF), jnp.float32)
                   for e in etypes])
    scratch_types = [
        pltpu.VMEM_SHARED((tot, HALF), jnp.float32),
        pltpu.VMEM((_CHUNK,), jnp.int32),
        pltpu.VMEM((_CHUNK,), jnp.int32),
        pltpu.VMEM((_CHUNK,), jnp.int32),
        pltpu.VMEM((_CHUNK,), jnp.float32),
        pltpu.VMEM((_CHUNK,), jnp.int32),
        pltpu.VMEM((_CHUNK,), jnp.int32),
        pltpu.VMEM((_CHUNK,), jnp.float32),
        pltpu.VMEM((_CHUNK, 16), jnp.float32),
        pltpu.VMEM((_CHUNK, HALF), jnp.float32),
        pltpu.VMEM((_CHUNK, HALF), jnp.float32),
        pltpu.SemaphoreType.DMA,
        pltpu.SemaphoreType.DMA,
    ]

    @functools.partial(pl.kernel, mesh=mesh, out_type=out_type,
                       scratch_types=scratch_types)
    def run(*refs):
        it = iter(refs)
        table_refs = [next(it) for _ in tables]
        ed_refs = [(next(it), next(it), next(it)) for _ in range(n_et)]
        s_refs = [next(it) for _ in range(n_et)]
        c2_refs = [next(it) for _ in range(n_et)]
        (acc, sidx, gidx0, didx0, ewv0, gidx1, didx1, ewv1, w16,
         rows0, rows1, sem0, sem1) = [next(it) for _ in range(13)]
        gidx, didx, ewv, rows = gidx0, didx0, ewv0, rows0
        slots = ((gidx0, didx0, ewv0, rows0, sem0),
                 (gidx1, didx1, ewv1, rows1, sem1))

        c = lax.axis_index("c")
        s = lax.axis_index("s")
        wid = c * _NT + s

        def zero_acc():
            def zbuf_body(i, _):
                zero16 = jnp.zeros((16,), jnp.float32)
                for j in range(HALF // 16):
                    rows0[i, pl.ds(j * 16, 16)] = zero16
                    rows1[i, pl.ds(j * 16, 16)] = zero16
                return 0

            lax.fori_loop(0, _CHUNK, zbuf_body, 0)

            def zacc_body(t, _):
                r0 = s * nz + t * _CHUNK
                pltpu.sync_copy(rows, acc.at[pl.ds(r0, _CHUNK)])
                return 0

            lax.fori_loop(0, nz // _CHUNK, zacc_body, 0)

        zero_acc()
        plsc.subcore_barrier()

        def load_idx(slot, gch, src_r, dst_r, ew_r, off, ept):
            gi, di, ev, _, _ = slot
            eb = s * ept + gch * _CHUNK
            pltpu.sync_copy(src_r.at[pl.ds(eb, _CHUNK)], sidx)
            pltpu.sync_copy(dst_r.at[pl.ds(eb, _CHUNK)], di)
            pltpu.sync_copy(ew_r.at[pl.ds(eb, _CHUNK)], ev)

            def idx_body(j, _):
                sl = pl.ds(j * 16, 16)
                gi[sl] = sidx[sl] * 2 + c
                di[sl] = di[sl] + off
                return 0

            lax.fori_loop(0, _CHUNK // 16, idx_body, 0)

        def start_gather(slot, tab_r):
            gi, _, _, rr, sm = slot
            pltpu.make_async_copy(tab_r.at[gi], rr, sm).start()

        def process(slot, tab_r):
            gi, di, ev, rr, sm = slot
            pltpu.make_async_copy(tab_r.at[gi], rr, sm).wait()

            def wexp_body(g2, _):
                wv = ev[pl.ds(g2 * 16, 16)]
                for k in range(16):
                    w16[g2 * 16 + k, :] = jnp.broadcast_to(wv[k], (16,))
                return 0

            lax.fori_loop(0, _CHUNK // 16, wexp_body, 0)

            def mul_body(i, _):
                wv = w16[i, :]
                for j in range(HALF // 16):
                    sl = pl.ds(j * 16, 16)
                    rr[i, sl] = rr[i, sl] * wv
                return 0

            lax.fori_loop(0, _CHUNK, mul_body, 0)
            pltpu.sync_copy(rr, acc.at[di], add=True)

        for idx_et in range(n_et):
            src_r, dst_r, ew_r = ed_refs[idx_et]
            tab_r = table_refs[etypes[idx_et]["ti"]]
            off = offs[idx_et]
            ept = etypes[idx_et]["src"].shape[0] // _NT
            nchunks = ept // _CHUNK

            load_idx(slots[0], 0, src_r, dst_r, ew_r, off, ept)
            start_gather(slots[0], tab_r)

            def pair_body(gg, _, src_r=src_r, dst_r=dst_r, ew_r=ew_r,
                          tab_r=tab_r, off=off, ept=ept, nchunks=nchunks):
                load_idx(slots[1], 2 * gg + 1, src_r, dst_r, ew_r, off, ept)
                start_gather(slots[1], tab_r)
                process(slots[0], tab_r)
                g2 = jnp.minimum(2 * gg + 2, nchunks - 1)
                load_idx(slots[0], g2, src_r, dst_r, ew_r, off, ept)
                start_gather(slots[0], tab_r)
                process(slots[1], tab_r)
                return 0

            lax.fori_loop(0, nchunks // 2, pair_body, 0)
            pltpu.make_async_copy(tab_r.at[gidx0], rows0, sem0).wait()

        plsc.subcore_barrier()

        for idx_et in range(n_et):
            nrf = etypes[idx_et]["ndst"] // _NT
            off = offs[idx_et]
            s_r = s_refs[idx_et]

            def flush_body(t, _, nrf=nrf, off=off, s_r=s_r):
                r0 = s * nrf + t * _CHUNK
                pltpu.sync_copy(acc.at[pl.ds(off + r0, _CHUNK)], rows)
                pltpu.sync_copy(
                    rows, s_r.at[pl.ds(r0, _CHUNK), pl.ds(c * HALF, HALF)])
                return 0

            lax.fori_loop(0, nrf // _CHUNK, flush_body, 0)

        plsc.subcore_barrier()
        zero_acc()
        plsc.subcore_barrier()

        def cload(slot, gch, dst_r, ew_r, off, ept2):
            _, di, ev, _, _ = slot
            eb = wid * ept2 + gch * _CHUNK
            pltpu.sync_copy(dst_r.at[pl.ds(eb, _CHUNK)], di)
            pltpu.sync_copy(ew_r.at[pl.ds(eb, _CHUNK)], ev)

            def idx_body(j, _):
                sl = pl.ds(j * 16, 16)
                di[sl] = di[sl] + off
                return 0

            lax.fori_loop(0, _CHUNK // 16, idx_body, 0)

        def cbuild(slot):
            _, _, ev, rr, _ = slot

            def cexp_body(g2, _):
                lane = lax.iota(jnp.int32, 16)
                sel0 = (1 - jnp.minimum(jnp.abs(lane), 1)).astype(jnp.float32)
                sel1 = (1 - jnp.minimum(jnp.abs(lane - 1), 1)).astype(jnp.float32)
                wv = ev[pl.ds(g2 * 16, 16)]
                for k in range(16):
                    wsp = jnp.broadcast_to(wv[k], (16,))
                    rr[g2 * 16 + k, pl.ds(0, 16)] = wsp * sel0 + sel1
                return 0

            lax.fori_loop(0, _CHUNK // 16, cexp_body, 0)

        def cwait(slot):
            _, di, _, rr, sm = slot
            pltpu.make_async_copy(rr, acc.at[di], sm).wait()

        for idx_et in range(n_et):
            _, dst_r, ew_r = ed_refs[idx_et]
            off = offs[idx_et]
            ept2 = etypes[idx_et]["src"].shape[0] // (2 * _NT)
            nchunks2 = ept2 // _CHUNK

            def cpair_body(gg, _, dst_r=dst_r, ew_r=ew_r, off=off, ept2=ept2):
                @pl.when(gg > 0)
                def _():
                    cwait(slots[0])
                    cwait(slots[1])

                for b in range(2):
                    slot = slots[b]
                    cload(slot, 2 * gg + b, dst_r, ew_r, off, ept2)
                    cbuild(slot)
                    pltpu.async_copy(slot[3], acc.at[slot[1]], slot[4],
                                     add=True)
                return 0

            lax.fori_loop(0, nchunks2 // 2, cpair_body, 0)
            cwait(slots[0])
            cwait(slots[1])

        plsc.subcore_barrier()

        for idx_et in range(n_et):
            nrf = etypes[idx_et]["ndst"] // _NT
            off = offs[idx_et]
            c2_r = c2_refs[idx_et]

            def cflush_body(t, _, nrf=nrf, off=off, c2_r=c2_r):
                r0 = s * nrf + t * _CHUNK
                pltpu.sync_copy(acc.at[pl.ds(off + r0, _CHUNK)], rows)
                pltpu.sync_copy(
                    rows, c2_r.at[pl.ds(r0, _CHUNK), pl.ds(c * HALF, HALF)])
                return 0

            lax.fori_loop(0, nrf // _CHUNK, cflush_body, 0)

    args = list(tables)
    for e in etypes:
        args += [e["src"], e["dst"], e["ew"]]
    outs = run(*args)
    return outs[:n_et], outs[n_et:]


def _scale_part(S, c2, W, b):
    wsum = c2[:, 0] + c2[:, HALF]
    cnt = jnp.maximum(c2[:, 1] + c2[:, HALF + 1], 1.0)
    inv = 1.0 / cnt
    h = jnp.dot(S * inv[:, None], W, preferred_element_type=jnp.float32)
    return h + (wsum * inv)[:, None] * b[None, :]


def _tc_combine1(S, c2, W, b):
    def body(s_ref, c2_ref, w_ref, b_ref, o_ref):
        o_ref[...] = _scale_part(s_ref[...], c2_ref[...], w_ref[...], b_ref[...])

    return pl.pallas_call(
        body, out_shape=jax.ShapeDtypeStruct((S.shape[0], IN_SIZE), jnp.float32),
    )(S, c2, W, b)


def _tc_combine2(Sa, c2a, Wa, ba, Sb, c2b, Wb, bb):
    def body(sa, ca, wa, ba_, sb, cb, wb, bb_, o_ref):
        o_ref[...] = (_scale_part(sa[...], ca[...], wa[...], ba_[...])
                      + _scale_part(sb[...], cb[...], wb[...], bb_[...]))

    return pl.pallas_call(
        body, out_shape=jax.ShapeDtypeStruct((Sa.shape[0], IN_SIZE), jnp.float32),
    )(Sa, c2a, Wa, ba, Sb, c2b, Wb, bb)


def kernel(feat_word, feat_topic, feat_doc,
           ei_ww, ei_wt, ei_wd, ei_td, ei_tt,
           w_ww, w_wt, w_wd, w_td, w_tt,
           W_ww, b_ww, W_wt, b_wt, W_wd, b_wd, W_td, b_td, W_tt, b_tt):
    word2 = feat_word.reshape(-1, HALF)
    topic2 = feat_topic.reshape(-1, HALF)

    s_ww, d_ww, e_ww = _prep_edges(ei_ww, w_ww, N_WORD)
    s_wt, d_wt, e_wt = _prep_edges(ei_wt, w_wt, N_TOPIC)
    s_wd, d_wd, e_wd = _prep_edges(ei_wd, w_wd, N_DOC)
    s_td, d_td, e_td = _prep_edges(ei_td, w_td, N_DOC)
    s_tt, d_tt, e_tt = _prep_edges(ei_tt, w_tt, N_TOPIC)

    (S_a, C_a) = _sc_segsum(
        [word2, topic2],
        [dict(ti=0, src=s_ww, dst=d_ww, ew=e_ww, ndst=N_WORD_P),
         dict(ti=1, src=s_tt, dst=d_tt, ew=e_tt, ndst=N_TOPIC_P)])
    S_ww_sum, S_tt_sum = S_a
    c2_ww, c2_tt = C_a

    h_word_p = _tc_combine1(S_ww_sum, c2_ww, W_ww, b_ww)
    hword2 = h_word_p.reshape(-1, HALF)
    h_word = h_word_p[:N_WORD]

    (S_c, C_c) = _sc_segsum(
        [hword2, topic2],
        [dict(ti=0, src=s_wt, dst=d_wt, ew=e_wt, ndst=N_TOPIC_P),
         dict(ti=0, src=s_wd, dst=d_wd, ew=e_wd, ndst=N_DOC_P),
         dict(ti=1, src=s_td, dst=d_td, ew=e_td, ndst=N_DOC_P)])
    S_wt_sum, S_wd_sum, S_td_sum = S_c
    c2_wt, c2_wd, c2_td = C_c

    h_topic = _tc_combine2(S_wt_sum, c2_wt, W_wt, b_wt,
                           S_tt_sum, c2_tt, W_tt, b_tt)[:N_TOPIC]
    h_doc = _tc_combine2(S_wd_sum, c2_wd, W_wd, b_wd,
                         S_td_sum, c2_td, W_td, b_td)[:N_DOC]
    return (h_word, h_topic, h_doc)
